# baseline (speedup 1.0000x reference)
import dataclasses
import functools

import jax
import jax.numpy as jnp
from jax import lax
from jax.experimental import pallas as pl
from jax.experimental.pallas import tpu as pltpu
from jax.experimental.pallas import tpu_sc as plsc

NUM_CORES = 2
NUM_SUBCORES = 16
LANES = 16
NUM_WORKERS = NUM_CORES * NUM_SUBCORES

BATCH = 16384
EMBED = 128
NUM_TRACK_ROWS = 100000
BPW = BATCH // NUM_WORKERS
CHUNK = 128
NCHUNKS = BPW // CHUNK


def _dot_sigmoid_kernel(tidx_hbm, nidx_hbm, temb_hbm, nemb_hbm,
                        tb_hbm, nb_hbm, out_hbm,
                        tidx_v, nidx_v,
                        trows0, nrows0, tb0, nb0, st0,
                        trows1, nrows1, tb1, nb1, st1,
                        sem0, sem1):
  wid = lax.axis_index("s") * NUM_CORES + lax.axis_index("c")
  base = wid * BPW

  pltpu.sync_copy(tidx_hbm.at[pl.ds(base, BPW)], tidx_v)
  pltpu.sync_copy(nidx_hbm.at[pl.ds(base, BPW)], nidx_v)
  lane = lax.iota(jnp.int32, LANES)

  bufs = [(trows0, nrows0, tb0, nb0), (trows1, nrows1, tb1, nb1)]
  stage = [st0, st1]
  sems = [sem0, sem1]

  def copies(c, p):
    i_t = tidx_v.at[pl.ds(c * CHUNK, CHUNK)]
    i_n = nidx_v.at[pl.ds(c * CHUNK, CHUNK)]
    trows, nrows, tbv, nbv = bufs[p]
    sem = sems[p]
    return (pltpu.make_async_copy(temb_hbm.at[i_t], trows, sem),
            pltpu.make_async_copy(nemb_hbm.at[i_n], nrows, sem),
            pltpu.make_async_copy(tb_hbm.at[i_t], tbv, sem),
            pltpu.make_async_copy(nb_hbm.at[i_n], nbv, sem))

  def fire(c, p):
    for d in copies(c, p):
      d.start()

  def drain(c, p):
    for d in copies(c, p):
      d.wait()

  def compute(c, p):
    trows, nrows, tbv, nbv = bufs[p]
    for g in range(CHUNK // LANES):
      dots = jnp.zeros((LANES,), jnp.float32)
      for r in range(LANES):
        row = g * LANES + r
        acc = trows[row, pl.ds(0, LANES)] * nrows[row, pl.ds(0, LANES)]
        for k in range(1, EMBED // LANES):
          acc = acc + (trows[row, pl.ds(k * LANES, LANES)] *
                       nrows[row, pl.ds(k * LANES, LANES)])
        dots = jnp.where(lane == r, jnp.sum(acc), dots)
      xv = dots + tbv[pl.ds(g * LANES, LANES)] + nbv[pl.ds(g * LANES, LANES)]
      yv = 1.0 / (1.0 + jnp.exp(-xv))
      stage[p][pl.ds(g * LANES, LANES)] = yv
    pltpu.sync_copy(stage[p], out_hbm.at[pl.ds(base + c * CHUNK, CHUNK)])

  fire(0, 0)
  fire(1, 1)

  @pl.loop(0, NCHUNKS, step=2)
  def _(c):
    drain(c, 0)
    compute(c, 0)

    @pl.when(c + 2 < NCHUNKS)
    def _():
      fire(c + 2, 0)

    drain(c + 1, 1)
    compute(c + 1, 1)

    @pl.when(c + 3 < NCHUNKS)
    def _():
      fire(c + 3, 1)


@jax.jit
def _run(tidx, nidx, temb, nemb, tb, nb):
  mesh = plsc.VectorSubcoreMesh(core_axis_name="c", subcore_axis_name="s")
  cp = pltpu.CompilerParams()
  if "needs_layout_passes" in pltpu.CompilerParams.__dataclass_fields__:
    cp = dataclasses.replace(cp, needs_layout_passes=False)
  row_bufs = [pltpu.VMEM((CHUNK, EMBED), jnp.float32),
              pltpu.VMEM((CHUNK, EMBED), jnp.float32),
              pltpu.VMEM((CHUNK,), jnp.float32),
              pltpu.VMEM((CHUNK,), jnp.float32),
              pltpu.VMEM((CHUNK,), jnp.float32)]
  kern = pl.kernel(
      _dot_sigmoid_kernel,
      out_type=jax.ShapeDtypeStruct((BATCH,), jnp.float32),
      mesh=mesh,
      scratch_types=(
          [pltpu.VMEM((BPW,), jnp.int32),
           pltpu.VMEM((BPW,), jnp.int32)]
          + row_bufs + row_bufs
          + [pltpu.SemaphoreType.DMA,
             pltpu.SemaphoreType.DMA]
      ),
      compiler_params=cp,
  )
  return kern(tidx, nidx, temb, nemb, tb, nb)


def kernel(inputs, track_embedding, name_embedding, track_bias, name_bias):
  idx = inputs.astype(jnp.int32)
  return _run(idx[:, 0], idx[:, 1], track_embedding, name_embedding,
              track_bias.reshape(-1), name_bias.reshape(-1))


# speedup vs baseline: 1.1357x; 1.1357x over previous
import dataclasses
import functools

import jax
import jax.numpy as jnp
from jax import lax
from jax.experimental import pallas as pl
from jax.experimental.pallas import tpu as pltpu
from jax.experimental.pallas import tpu_sc as plsc

NUM_CORES = 2
NUM_SUBCORES = 16
LANES = 16
NUM_WORKERS = NUM_CORES * NUM_SUBCORES

BATCH = 16384
EMBED = 128
NUM_TRACK_ROWS = 100000
BPW = BATCH // NUM_WORKERS
CHUNK = 64
NCHUNKS = BPW // CHUNK


def _dot_sigmoid_kernel(tidx_hbm, nidx_hbm, temb_hbm, nemb_hbm,
                        tb_hbm, nb_hbm, out_hbm,
                        tidx_v, nidx_v,
                        trows0, nrows0, tb0, nb0, st0,
                        trows1, nrows1, tb1, nb1, st1,
                        sem0, sem1):
  wid = lax.axis_index("s") * NUM_CORES + lax.axis_index("c")
  base = wid * BPW

  pltpu.sync_copy(tidx_hbm.at[pl.ds(base, BPW)], tidx_v)
  pltpu.sync_copy(nidx_hbm.at[pl.ds(base, BPW)], nidx_v)
  lane = lax.iota(jnp.int32, LANES)

  bufs = [(trows0, nrows0, tb0, nb0), (trows1, nrows1, tb1, nb1)]
  stage = [st0, st1]
  sems = [sem0, sem1]

  def copies(c, p):
    i_t = tidx_v.at[pl.ds(c * CHUNK, CHUNK)]
    i_n = nidx_v.at[pl.ds(c * CHUNK, CHUNK)]
    trows, nrows, tbv, nbv = bufs[p]
    sem = sems[p]
    return (pltpu.make_async_copy(temb_hbm.at[i_t], trows, sem),
            pltpu.make_async_copy(nemb_hbm.at[i_n], nrows, sem),
            pltpu.make_async_copy(tb_hbm.at[i_t], tbv, sem),
            pltpu.make_async_copy(nb_hbm.at[i_n], nbv, sem))

  def fire(c, p):
    for d in copies(c, p):
      d.start()

  def drain(c, p):
    for d in copies(c, p):
      d.wait()

  def compute(c, p):
    trows, nrows, tbv, nbv = bufs[p]
    for g in range(CHUNK // LANES):
      dots = jnp.zeros((LANES,), jnp.float32)
      for r in range(LANES):
        row = g * LANES + r
        acc = trows[row, pl.ds(0, LANES)] * nrows[row, pl.ds(0, LANES)]
        for k in range(1, EMBED // LANES):
          acc = acc + (trows[row, pl.ds(k * LANES, LANES)] *
                       nrows[row, pl.ds(k * LANES, LANES)])
        dots = jnp.where(lane == r, jnp.sum(acc), dots)
      xv = dots + tbv[pl.ds(g * LANES, LANES)] + nbv[pl.ds(g * LANES, LANES)]
      yv = 1.0 / (1.0 + jnp.exp(-xv))
      stage[p][pl.ds(g * LANES, LANES)] = yv
    pltpu.sync_copy(stage[p], out_hbm.at[pl.ds(base + c * CHUNK, CHUNK)])

  fire(0, 0)
  fire(1, 1)

  @pl.loop(0, NCHUNKS, step=2)
  def _(c):
    drain(c, 0)
    compute(c, 0)

    @pl.when(c + 2 < NCHUNKS)
    def _():
      fire(c + 2, 0)

    drain(c + 1, 1)
    compute(c + 1, 1)

    @pl.when(c + 3 < NCHUNKS)
    def _():
      fire(c + 3, 1)


@jax.jit
def _run(tidx, nidx, temb, nemb, tb, nb):
  mesh = plsc.VectorSubcoreMesh(core_axis_name="c", subcore_axis_name="s")
  cp = pltpu.CompilerParams()
  if "needs_layout_passes" in pltpu.CompilerParams.__dataclass_fields__:
    cp = dataclasses.replace(cp, needs_layout_passes=False)
  row_bufs = [pltpu.VMEM((CHUNK, EMBED), jnp.float32),
              pltpu.VMEM((CHUNK, EMBED), jnp.float32),
              pltpu.VMEM((CHUNK,), jnp.float32),
              pltpu.VMEM((CHUNK,), jnp.float32),
              pltpu.VMEM((CHUNK,), jnp.float32)]
  kern = pl.kernel(
      _dot_sigmoid_kernel,
      out_type=jax.ShapeDtypeStruct((BATCH,), jnp.float32),
      mesh=mesh,
      scratch_types=(
          [pltpu.VMEM((BPW,), jnp.int32),
           pltpu.VMEM((BPW,), jnp.int32)]
          + row_bufs + row_bufs
          + [pltpu.SemaphoreType.DMA,
             pltpu.SemaphoreType.DMA]
      ),
      compiler_params=cp,
  )
  return kern(tidx, nidx, temb, nemb, tb, nb)


def kernel(inputs, track_embedding, name_embedding, track_bias, name_bias):
  idx = inputs.astype(jnp.int32)
  return _run(idx[:, 0], idx[:, 1], track_embedding, name_embedding,
              track_bias.reshape(-1), name_bias.reshape(-1))


# speedup vs baseline: 1.2519x; 1.1023x over previous
import dataclasses
import functools

import jax
import jax.numpy as jnp
from jax import lax
from jax.experimental import pallas as pl
from jax.experimental.pallas import tpu as pltpu
from jax.experimental.pallas import tpu_sc as plsc

NUM_CORES = 2
NUM_SUBCORES = 16
LANES = 16
NUM_WORKERS = NUM_CORES * NUM_SUBCORES

BATCH = 16384
EMBED = 128
NUM_TRACK_ROWS = 100000
BPW = BATCH // NUM_WORKERS
CHUNK = 64
NCHUNKS = BPW // CHUNK


def _dot_sigmoid_kernel(tidx_hbm, nidx_hbm, temb_hbm, nemb_hbm,
                        tb_hbm, nb_hbm, out_hbm,
                        tidx_v, nidx_v,
                        trows0, nrows0, tb0, nb0,
                        trows1, nrows1, tb1, nb1,
                        out_v, sem0, sem1):
  wid = lax.axis_index("s") * NUM_CORES + lax.axis_index("c")
  base = wid * BPW

  pltpu.sync_copy(tidx_hbm.at[pl.ds(base, BPW)], tidx_v)
  pltpu.sync_copy(nidx_hbm.at[pl.ds(base, BPW)], nidx_v)
  lane = lax.iota(jnp.int32, LANES)

  bufs = [(trows0, nrows0, tb0, nb0), (trows1, nrows1, tb1, nb1)]
  sems = [sem0, sem1]

  def copies(c, p):
    i_t = tidx_v.at[pl.ds(c * CHUNK, CHUNK)]
    i_n = nidx_v.at[pl.ds(c * CHUNK, CHUNK)]
    trows, nrows, tbv, nbv = bufs[p]
    sem = sems[p]
    return (pltpu.make_async_copy(temb_hbm.at[i_t], trows, sem),
            pltpu.make_async_copy(nemb_hbm.at[i_n], nrows, sem),
            pltpu.make_async_copy(tb_hbm.at[i_t], tbv, sem),
            pltpu.make_async_copy(nb_hbm.at[i_n], nbv, sem))

  def fire(c, p):
    for d in copies(c, p):
      d.start()

  def drain(c, p):
    for d in copies(c, p):
      d.wait()

  def compute(c, p):
    trows, nrows, tbv, nbv = bufs[p]
    for g in range(CHUNK // LANES):
      dots = jnp.zeros((LANES,), jnp.float32)
      for r in range(LANES):
        row = g * LANES + r
        acc = trows[row, pl.ds(0, LANES)] * nrows[row, pl.ds(0, LANES)]
        for k in range(1, EMBED // LANES):
          acc = acc + (trows[row, pl.ds(k * LANES, LANES)] *
                       nrows[row, pl.ds(k * LANES, LANES)])
        dots = jnp.where(lane == r, jnp.sum(acc), dots)
      xv = dots + tbv[pl.ds(g * LANES, LANES)] + nbv[pl.ds(g * LANES, LANES)]
      yv = 1.0 / (1.0 + jnp.exp(-xv))
      out_v[pl.ds(c * CHUNK + g * LANES, LANES)] = yv

  fire(0, 0)
  fire(1, 1)

  @pl.loop(0, NCHUNKS, step=2)
  def _(c):
    drain(c, 0)
    compute(c, 0)

    @pl.when(c + 2 < NCHUNKS)
    def _():
      fire(c + 2, 0)

    drain(c + 1, 1)
    compute(c + 1, 1)

    @pl.when(c + 3 < NCHUNKS)
    def _():
      fire(c + 3, 1)

  pltpu.sync_copy(out_v, out_hbm.at[pl.ds(base, BPW)])


@jax.jit
def _run(tidx, nidx, temb, nemb, tb, nb):
  mesh = plsc.VectorSubcoreMesh(core_axis_name="c", subcore_axis_name="s")
  cp = pltpu.CompilerParams()
  if "needs_layout_passes" in pltpu.CompilerParams.__dataclass_fields__:
    cp = dataclasses.replace(cp, needs_layout_passes=False)
  row_bufs = [pltpu.VMEM((CHUNK, EMBED), jnp.float32),
              pltpu.VMEM((CHUNK, EMBED), jnp.float32),
              pltpu.VMEM((CHUNK,), jnp.float32),
              pltpu.VMEM((CHUNK,), jnp.float32)]
  kern = pl.kernel(
      _dot_sigmoid_kernel,
      out_type=jax.ShapeDtypeStruct((BATCH,), jnp.float32),
      mesh=mesh,
      scratch_types=(
          [pltpu.VMEM((BPW,), jnp.int32),
           pltpu.VMEM((BPW,), jnp.int32)]
          + row_bufs + row_bufs
          + [pltpu.VMEM((BPW,), jnp.float32),
             pltpu.SemaphoreType.DMA,
             pltpu.SemaphoreType.DMA]
      ),
      compiler_params=cp,
  )
  return kern(tidx, nidx, temb, nemb, tb, nb)


def kernel(inputs, track_embedding, name_embedding, track_bias, name_bias):
  idx = inputs.astype(jnp.int32)
  return _run(idx[:, 0], idx[:, 1], track_embedding, name_embedding,
              track_bias.reshape(-1), name_bias.reshape(-1))
